# Initial kernel scaffold; baseline (speedup 1.0000x reference)
import jax
import jax.numpy as jnp
from jax.experimental import pallas as pl


def kernel(x, edge_index, edge_attr, u, batch, W1, b1, g1, be1, W2, b2, g2, be2):
    raise NotImplementedError("write your pallas kernel here")


# speedup vs baseline: 2.1775x; 2.1775x over previous
import functools
import jax
import jax.numpy as jnp
from jax import lax
from jax.experimental import pallas as pl
from jax.experimental.pallas import tpu as pltpu
from jax.experimental.pallas import tpu_sc as plsc

N = 10000
E = 320000
C = 128
EPS = 1e-4

NC = 2
NS = 16
NW = NC * NS
EW = E // NW
K = 80
NCHUNK = EW // K
NP_ = 10240
STRIPE = NP_ // NS

BE = 512
NBLK = E // BE


def _pre_body(x_ref, w1_ref, b1_ref, w2_ref, t1_ref, t2_ref):
    x = x_ref[...]
    dn = (((1,), (1,)), ((), ()))
    t1_ref[...] = (
        lax.dot_general(x, w1_ref[:, :C], dn, preferred_element_type=jnp.float32)
        + b1_ref[...]
    )
    t2_ref[...] = lax.dot_general(
        x, w2_ref[:, :C], dn, preferred_element_type=jnp.float32
    )


def _pre(x, W1, b1, W2):
    return pl.pallas_call(
        _pre_body,
        out_shape=[
            jax.ShapeDtypeStruct((N, C), jnp.float32),
            jax.ShapeDtypeStruct((N, C), jnp.float32),
        ],
    )(x, W1, b1, W2)


_MESH = plsc.VectorSubcoreMesh(core_axis_name="c", subcore_axis_name="s")


@functools.partial(
    pl.kernel,
    out_type=jax.ShapeDtypeStruct((E, C), jnp.float32),
    mesh=_MESH,
    scratch_types=[
        pltpu.VMEM((EW,), jnp.int32),
        pltpu.VMEM((K, C), jnp.float32),
        pltpu.SemaphoreType.DMA,
    ],
    compiler_params=pltpu.CompilerParams(needs_layout_passes=False),
)
def _gather(t1_hbm, row_hbm, g_hbm, idx_v, rows_v, sem):
    cid = lax.axis_index("c")
    sid = lax.axis_index("s")
    wid = sid * NC + cid
    base = wid * EW
    pltpu.sync_copy(row_hbm.at[pl.ds(base, EW)], idx_v)

    def step(j, carry):
        off = j * K
        pltpu.async_copy(t1_hbm.at[idx_v.at[pl.ds(off, K)]], rows_v, sem).wait()
        pltpu.sync_copy(rows_v, g_hbm.at[pl.ds(base + off, K)])
        return carry

    lax.fori_loop(0, NCHUNK, step, 0)


def _edge_body(g_ref, ea_ref, w1_ref, h_ref, st_ref):
    i = pl.program_id(0)
    dn = (((1,), (1,)), ((), ()))
    a = lax.dot_general(
        ea_ref[...], w1_ref[:, C:], dn, preferred_element_type=jnp.float32
    )
    h = a + g_ref[...]
    h_ref[...] = h
    s = jnp.sum(h, axis=0)
    q = jnp.sum(h * h, axis=0)
    upd = jnp.concatenate(
        [s[None], q[None], jnp.zeros((6, C), jnp.float32)], axis=0
    )

    @pl.when(i == 0)
    def _():
        st_ref[...] = upd

    @pl.when(i > 0)
    def _():
        st_ref[...] = st_ref[...] + upd


def _edge(G, edge_attr, W1):
    return pl.pallas_call(
        _edge_body,
        grid=(NBLK,),
        in_specs=[
            pl.BlockSpec((BE, C), lambda i: (i, 0)),
            pl.BlockSpec((BE, C), lambda i: (i, 0)),
            pl.BlockSpec((C, 2 * C), lambda i: (0, 0)),
        ],
        out_specs=[
            pl.BlockSpec((BE, C), lambda i: (i, 0)),
            pl.BlockSpec((8, C), lambda i: (0, 0)),
        ],
        out_shape=[
            jax.ShapeDtypeStruct((E, C), jnp.float32),
            jax.ShapeDtypeStruct((8, C), jnp.float32),
        ],
    )(G, edge_attr, W1)


@functools.partial(
    pl.kernel,
    out_type=[
        jax.ShapeDtypeStruct((NC, NP_, C), jnp.float32),
        jax.ShapeDtypeStruct((NW, N), jnp.float32),
    ],
    mesh=_MESH,
    scratch_types=[
        pltpu.VMEM((NCHUNK, K), jnp.int32),
        pltpu.VMEM((K, C), jnp.float32),
        pltpu.VMEM((N,), jnp.float32),
        pltpu.VMEM((C,), jnp.float32),
        pltpu.VMEM((C,), jnp.float32),
        pltpu.VMEM_SHARED((NP_, C), jnp.float32),
    ],
    compiler_params=pltpu.CompilerParams(needs_layout_passes=False),
)
def _scatter(h_hbm, col3_hbm, sc_hbm, sh_hbm, zrow_hbm, zcnt_hbm,
             ssum_hbm, cnt_hbm,
             col2d, hbuf, cntbuf, scbuf, shbuf, accum):
    cid = lax.axis_index("c")
    sid = lax.axis_index("s")
    wid = sid * NC + cid
    base = wid * EW

    pltpu.sync_copy(col3_hbm.at[wid], col2d)
    pltpu.sync_copy(sc_hbm, scbuf)
    pltpu.sync_copy(sh_hbm, shbuf)
    pltpu.sync_copy(zcnt_hbm, cntbuf)
    pltpu.sync_copy(zrow_hbm, accum.at[pl.ds(sid * STRIPE, STRIPE)])
    plsc.subcore_barrier()

    svs = [scbuf[pl.ds(g * 16, 16)] for g in range(8)]
    shs = [shbuf[pl.ds(g * 16, 16)] for g in range(8)]
    ones16 = jnp.full((16,), 1.0, jnp.float32)

    def chunk(j, carry):
        pltpu.sync_copy(h_hbm.at[pl.ds(base + j * K, K)], hbuf)

        def row_step(r, c2):
            for g in range(8):
                hv = hbuf[r, pl.ds(g * 16, 16)]
                yv = hv * svs[g] + shs[g]
                yv = jnp.where(yv > 0.0, yv, jnp.exp(yv) - 1.0)
                hbuf[r, pl.ds(g * 16, 16)] = yv
            return c2

        lax.fori_loop(0, K, row_step, 0)

        def cnt_step(t, c2):
            cv = col2d[j, pl.ds(t * 16, 16)]
            plsc.addupdate_scatter(cntbuf, [cv], ones16)
            return c2

        lax.fori_loop(0, K // 16, cnt_step, 0)

        pltpu.sync_copy(hbuf, accum.at[col2d.at[j]], add=True)
        return carry

    lax.fori_loop(0, NCHUNK, chunk, 0)

    plsc.subcore_barrier()
    pltpu.sync_copy(
        accum.at[pl.ds(sid * STRIPE, STRIPE)],
        ssum_hbm.at[cid, pl.ds(sid * STRIPE, STRIPE)],
    )
    pltpu.sync_copy(cntbuf, cnt_hbm.at[wid])


def _node_body(t2_ref, ss_ref, cnt_ref, w2_ref, b2_ref, g2_ref, be2_ref, o_ref):
    ssum = ss_ref[0] + ss_ref[1]
    cnt = jnp.sum(cnt_ref[...], axis=0)
    mean = ssum / jnp.clip(cnt, 1.0, None)[:, None]
    dn = (((1,), (1,)), ((), ()))
    o = (
        lax.dot_general(mean, w2_ref[:, C:], dn, preferred_element_type=jnp.float32)
        + t2_ref[...]
        + b2_ref[...]
    )
    m = jnp.mean(o, axis=0, keepdims=True)
    v = jnp.mean(o * o, axis=0, keepdims=True) - m * m
    y = (o - m) * lax.rsqrt(v + EPS) * g2_ref[...] + be2_ref[...]
    o_ref[...] = jnp.where(y > 0.0, y, jnp.exp(y) - 1.0)


def _node(t2, ssum, cnt, W2, b2, g2, be2):
    return pl.pallas_call(
        _node_body,
        out_shape=jax.ShapeDtypeStruct((N, C), jnp.float32),
    )(t2, ssum, cnt, W2, b2, g2, be2)


def kernel(x, edge_index, edge_attr, u, batch, W1, b1, g1, be1, W2, b2, g2, be2):
    row = edge_index[0]
    col3 = edge_index[1].reshape(NW, NCHUNK, K)
    t1, t2 = _pre(x, W1, b1.reshape(1, C), W2)
    G = _gather(t1, row)
    h, st = _edge(G, edge_attr, W1)
    m = st[0] / E
    var = st[1] / E - m * m
    scale = g1 / jnp.sqrt(var + EPS)
    shift = be1 - m * scale
    zrow = jnp.zeros((STRIPE, C), jnp.float32)
    zcnt = jnp.zeros((N,), jnp.float32)
    ssum, cnt = _scatter(h, col3, scale, shift, zrow, zcnt)
    ssum = ssum[:, :N]
    return _node(
        t2, ssum, cnt, W2, b2.reshape(1, C), g2.reshape(1, C), be2.reshape(1, C)
    )


# speedup vs baseline: 2.6868x; 1.2339x over previous
import functools
import jax
import jax.numpy as jnp
from jax import lax
from jax.experimental import pallas as pl
from jax.experimental.pallas import tpu as pltpu
from jax.experimental.pallas import tpu_sc as plsc

N = 10000
E = 320000
C = 128
EPS = 1e-4

NC = 2
NS = 16
NW = NC * NS
EW = E // NW
K = 80
NCHUNK = EW // K
NP_ = 10240
STRIPE = NP_ // NS

BE = 512
NBLK = E // BE


def _pre_body(x_ref, w1_ref, b1_ref, w2_ref, t1_ref, t2_ref):
    x = x_ref[...]
    dn = (((1,), (1,)), ((), ()))
    t1_ref[...] = (
        lax.dot_general(x, w1_ref[:, :C], dn, preferred_element_type=jnp.float32)
        + b1_ref[...]
    )
    t2_ref[...] = lax.dot_general(
        x, w2_ref[:, :C], dn, preferred_element_type=jnp.float32
    )


def _pre(x, W1, b1, W2):
    return pl.pallas_call(
        _pre_body,
        out_shape=[
            jax.ShapeDtypeStruct((N, C), jnp.float32),
            jax.ShapeDtypeStruct((N, C), jnp.float32),
        ],
    )(x, W1, b1, W2)


_MESH = plsc.VectorSubcoreMesh(core_axis_name="c", subcore_axis_name="s")


@functools.partial(
    pl.kernel,
    out_type=jax.ShapeDtypeStruct((E, C), jnp.float32),
    mesh=_MESH,
    scratch_types=[
        pltpu.VMEM((K, C), jnp.float32),
        pltpu.VMEM((K, C), jnp.float32),
        pltpu.VMEM((EW,), jnp.int32),
        pltpu.SemaphoreType.DMA,
        pltpu.SemaphoreType.DMA,
    ],
    compiler_params=pltpu.CompilerParams(needs_layout_passes=False),
)
def _gather(t1_hbm, row_hbm, g_hbm, rows_a, rows_b, idx_v, sem_g, sem_w):
    bufs = [rows_a, rows_b]
    cid = lax.axis_index("c")
    sid = lax.axis_index("s")
    wid = sid * NC + cid
    base = wid * EW
    pltpu.sync_copy(row_hbm.at[pl.ds(base, EW)], idx_v)

    def fire(j, b):
        pltpu.async_copy(t1_hbm.at[idx_v.at[pl.ds(j * K, K)]], bufs[b], sem_g)

    def slot(j, b, last):
        @pl.when(j >= 1)
        def _():
            pltpu.make_async_copy(bufs[1 - b], g_hbm.at[pl.ds(base, K)], sem_w).wait()

        if not last:
            fire(j + 1, 1 - b)
        pltpu.make_async_copy(
            t1_hbm.at[idx_v.at[pl.ds(0, K)]], bufs[b], sem_g
        ).wait()
        pltpu.async_copy(bufs[b], g_hbm.at[pl.ds(base + j * K, K)], sem_w)

    fire(jnp.int32(0), 0)

    def outer(j0):
        slot(j0, 0, False)
        slot(j0 + 1, 1, False)

    pl.loop(0, NCHUNK - 1, step=2)(outer)
    slot(jnp.int32(NCHUNK - 1), 0, True)
    pltpu.make_async_copy(bufs[0], g_hbm.at[pl.ds(base, K)], sem_w).wait()


def _edge_body(g_ref, ea_ref, w1_ref, h_ref, st_ref):
    i = pl.program_id(0)
    dn = (((1,), (1,)), ((), ()))
    a = lax.dot_general(
        ea_ref[...], w1_ref[:, C:], dn, preferred_element_type=jnp.float32
    )
    h = a + g_ref[...]
    h_ref[...] = h
    s = jnp.sum(h, axis=0)
    q = jnp.sum(h * h, axis=0)
    upd = jnp.concatenate(
        [s[None], q[None], jnp.zeros((6, C), jnp.float32)], axis=0
    )

    @pl.when(i == 0)
    def _():
        st_ref[...] = upd

    @pl.when(i > 0)
    def _():
        st_ref[...] = st_ref[...] + upd


def _edge(G, edge_attr, W1):
    return pl.pallas_call(
        _edge_body,
        grid=(NBLK,),
        in_specs=[
            pl.BlockSpec((BE, C), lambda i: (i, 0)),
            pl.BlockSpec((BE, C), lambda i: (i, 0)),
            pl.BlockSpec((C, 2 * C), lambda i: (0, 0)),
        ],
        out_specs=[
            pl.BlockSpec((BE, C), lambda i: (i, 0)),
            pl.BlockSpec((8, C), lambda i: (0, 0)),
        ],
        out_shape=[
            jax.ShapeDtypeStruct((E, C), jnp.float32),
            jax.ShapeDtypeStruct((8, C), jnp.float32),
        ],
    )(G, edge_attr, W1)


@functools.partial(
    pl.kernel,
    out_type=[
        jax.ShapeDtypeStruct((NC, NP_, C), jnp.float32),
        jax.ShapeDtypeStruct((NW, N), jnp.float32),
    ],
    mesh=_MESH,
    scratch_types=[
        pltpu.VMEM((NCHUNK, K), jnp.int32),
        pltpu.VMEM((K, C), jnp.float32),
        pltpu.VMEM((K, C), jnp.float32),
        pltpu.VMEM((N,), jnp.float32),
        pltpu.VMEM((C,), jnp.float32),
        pltpu.VMEM((C,), jnp.float32),
        pltpu.VMEM_SHARED((NP_, C), jnp.float32),
        pltpu.SemaphoreType.DMA,
    ],
    compiler_params=pltpu.CompilerParams(needs_layout_passes=False),
)
def _scatter(h_hbm, col3_hbm, sc_hbm, sh_hbm, zrow_hbm, zcnt_hbm,
             ssum_hbm, cnt_hbm,
             col2d, ha, hb, cntbuf, scbuf, shbuf, accum, sem_l):
    bufs = [ha, hb]
    cid = lax.axis_index("c")
    sid = lax.axis_index("s")
    wid = sid * NC + cid
    base = wid * EW

    pltpu.sync_copy(col3_hbm.at[wid], col2d)
    pltpu.sync_copy(sc_hbm, scbuf)
    pltpu.sync_copy(sh_hbm, shbuf)
    pltpu.sync_copy(zcnt_hbm, cntbuf)
    pltpu.sync_copy(zrow_hbm, accum.at[pl.ds(sid * STRIPE, STRIPE)])
    plsc.subcore_barrier()

    svs = [scbuf[pl.ds(g * 16, 16)] for g in range(8)]
    shs = [shbuf[pl.ds(g * 16, 16)] for g in range(8)]
    ones16 = jnp.full((16,), 1.0, jnp.float32)

    def fire(j, b):
        pltpu.async_copy(h_hbm.at[pl.ds(base + j * K, K)], bufs[b], sem_l)

    def slot(j, b, last):
        if not last:
            fire(j + 1, 1 - b)
        pltpu.make_async_copy(h_hbm.at[pl.ds(base, K)], bufs[b], sem_l).wait()

        def row_step(r, c2):
            for g in range(8):
                hv = bufs[b][r, pl.ds(g * 16, 16)]
                yv = hv * svs[g] + shs[g]
                yv = jnp.where(yv > 0.0, yv, jnp.exp(yv) - 1.0)
                bufs[b][r, pl.ds(g * 16, 16)] = yv
            return c2

        lax.fori_loop(0, K, row_step, 0)

        def cnt_step(t, c2):
            cv = col2d[j, pl.ds(t * 16, 16)]
            plsc.addupdate_scatter(cntbuf, [cv], ones16)
            return c2

        lax.fori_loop(0, K // 16, cnt_step, 0)

        pltpu.sync_copy(bufs[b], accum.at[col2d.at[j]], add=True)

    fire(jnp.int32(0), 0)

    def outer(j0):
        slot(j0, 0, False)
        slot(j0 + 1, 1, False)

    pl.loop(0, NCHUNK - 1, step=2)(outer)
    slot(jnp.int32(NCHUNK - 1), 0, True)

    plsc.subcore_barrier()
    pltpu.sync_copy(
        accum.at[pl.ds(sid * STRIPE, STRIPE)],
        ssum_hbm.at[cid, pl.ds(sid * STRIPE, STRIPE)],
    )
    pltpu.sync_copy(cntbuf, cnt_hbm.at[wid])


def _node_body(t2_ref, ss_ref, cnt_ref, w2_ref, b2_ref, g2_ref, be2_ref, o_ref):
    ssum = ss_ref[0] + ss_ref[1]
    cnt = jnp.sum(cnt_ref[...], axis=0)
    mean = ssum / jnp.clip(cnt, 1.0, None)[:, None]
    dn = (((1,), (1,)), ((), ()))
    o = (
        lax.dot_general(mean, w2_ref[:, C:], dn, preferred_element_type=jnp.float32)
        + t2_ref[...]
        + b2_ref[...]
    )
    m = jnp.mean(o, axis=0, keepdims=True)
    v = jnp.mean(o * o, axis=0, keepdims=True) - m * m
    y = (o - m) * lax.rsqrt(v + EPS) * g2_ref[...] + be2_ref[...]
    o_ref[...] = jnp.where(y > 0.0, y, jnp.exp(y) - 1.0)


def _node(t2, ssum, cnt, W2, b2, g2, be2):
    return pl.pallas_call(
        _node_body,
        out_shape=jax.ShapeDtypeStruct((N, C), jnp.float32),
    )(t2, ssum, cnt, W2, b2, g2, be2)


def kernel(x, edge_index, edge_attr, u, batch, W1, b1, g1, be1, W2, b2, g2, be2):
    row = edge_index[0]
    col3 = edge_index[1].reshape(NW, NCHUNK, K)
    t1, t2 = _pre(x, W1, b1.reshape(1, C), W2)
    G = _gather(t1, row)
    h, st = _edge(G, edge_attr, W1)
    m = st[0] / E
    var = st[1] / E - m * m
    scale = g1 / jnp.sqrt(var + EPS)
    shift = be1 - m * scale
    zrow = jnp.zeros((STRIPE, C), jnp.float32)
    zcnt = jnp.zeros((N,), jnp.float32)
    ssum, cnt = _scatter(h, col3, scale, shift, zrow, zcnt)
    ssum = ssum[:, :N]
    return _node(
        t2, ssum, cnt, W2, b2.reshape(1, C), g2.reshape(1, C), be2.reshape(1, C)
    )


# speedup vs baseline: 2.8240x; 1.0511x over previous
import functools
import jax
import jax.numpy as jnp
from jax import lax
from jax.experimental import pallas as pl
from jax.experimental.pallas import tpu as pltpu
from jax.experimental.pallas import tpu_sc as plsc

N = 10000
E = 320000
C = 128
EPS = 1e-4

NC = 2
NS = 16
NW = NC * NS
EW = E // NW
K = 80
NCHUNK = EW // K
NP_ = 10240
STRIPE = NP_ // NS

BE = 512
NBLK = E // BE


def _pre_body(x_ref, w1_ref, b1_ref, w2_ref, t1_ref, t2_ref):
    x = x_ref[...]
    dn = (((1,), (1,)), ((), ()))
    t1_ref[...] = (
        lax.dot_general(x, w1_ref[:, :C], dn, preferred_element_type=jnp.float32)
        + b1_ref[...]
    )
    t2_ref[...] = lax.dot_general(
        x, w2_ref[:, :C], dn, preferred_element_type=jnp.float32
    )


def _pre(x, W1, b1, W2):
    return pl.pallas_call(
        _pre_body,
        out_shape=[
            jax.ShapeDtypeStruct((N, C), jnp.float32),
            jax.ShapeDtypeStruct((N, C), jnp.float32),
        ],
    )(x, W1, b1, W2)


_MESH = plsc.VectorSubcoreMesh(core_axis_name="c", subcore_axis_name="s")


@functools.partial(
    pl.kernel,
    out_type=jax.ShapeDtypeStruct((E, C), jnp.float32),
    mesh=_MESH,
    scratch_types=[
        pltpu.VMEM((K, C), jnp.float32),
        pltpu.VMEM((K, C), jnp.float32),
        pltpu.VMEM((EW,), jnp.int32),
        pltpu.SemaphoreType.DMA,
        pltpu.SemaphoreType.DMA,
    ],
    compiler_params=pltpu.CompilerParams(needs_layout_passes=False),
)
def _gather(t1_hbm, row_hbm, g_hbm, rows_a, rows_b, idx_v, sem_g, sem_w):
    bufs = [rows_a, rows_b]
    cid = lax.axis_index("c")
    sid = lax.axis_index("s")
    wid = sid * NC + cid
    base = wid * EW
    pltpu.sync_copy(row_hbm.at[pl.ds(base, EW)], idx_v)

    def fire(j, b):
        pltpu.async_copy(t1_hbm.at[idx_v.at[pl.ds(j * K, K)]], bufs[b], sem_g)

    def slot(j, b, last):
        @pl.when(j >= 1)
        def _():
            pltpu.make_async_copy(bufs[1 - b], g_hbm.at[pl.ds(base, K)], sem_w).wait()

        if not last:
            fire(j + 1, 1 - b)
        pltpu.make_async_copy(
            t1_hbm.at[idx_v.at[pl.ds(0, K)]], bufs[b], sem_g
        ).wait()
        pltpu.async_copy(bufs[b], g_hbm.at[pl.ds(base + j * K, K)], sem_w)

    fire(jnp.int32(0), 0)

    def outer(j0):
        slot(j0, 0, False)
        slot(j0 + 1, 1, False)

    pl.loop(0, NCHUNK - 1, step=2)(outer)
    slot(jnp.int32(NCHUNK - 1), 0, True)
    pltpu.make_async_copy(bufs[0], g_hbm.at[pl.ds(base, K)], sem_w).wait()


def _edge_body(g_ref, ea_ref, w1_ref, h_ref, st_ref):
    i = pl.program_id(0)
    dn = (((1,), (1,)), ((), ()))
    a = lax.dot_general(
        ea_ref[...], w1_ref[:, C:], dn, preferred_element_type=jnp.float32
    )
    h = a + g_ref[...]
    h_ref[...] = h
    s = jnp.sum(h, axis=0)
    q = jnp.sum(h * h, axis=0)
    upd = jnp.concatenate(
        [s[None], q[None], jnp.zeros((6, C), jnp.float32)], axis=0
    )

    @pl.when(i == 0)
    def _():
        st_ref[...] = upd

    @pl.when(i > 0)
    def _():
        st_ref[...] = st_ref[...] + upd


def _edge(G, edge_attr, W1):
    return pl.pallas_call(
        _edge_body,
        grid=(NBLK,),
        in_specs=[
            pl.BlockSpec((BE, C), lambda i: (i, 0)),
            pl.BlockSpec((BE, C), lambda i: (i, 0)),
            pl.BlockSpec((C, 2 * C), lambda i: (0, 0)),
        ],
        out_specs=[
            pl.BlockSpec((BE, C), lambda i: (i, 0)),
            pl.BlockSpec((8, C), lambda i: (0, 0)),
        ],
        out_shape=[
            jax.ShapeDtypeStruct((E, C), jnp.float32),
            jax.ShapeDtypeStruct((8, C), jnp.float32),
        ],
    )(G, edge_attr, W1)


@functools.partial(
    pl.kernel,
    out_type=[
        jax.ShapeDtypeStruct((NC, NP_, C), jnp.float32),
        jax.ShapeDtypeStruct((NW, N), jnp.float32),
    ],
    mesh=_MESH,
    scratch_types=[
        pltpu.VMEM((NCHUNK, K), jnp.int32),
        pltpu.VMEM((K, C), jnp.float32),
        pltpu.VMEM((K, C), jnp.float32),
        pltpu.VMEM((N,), jnp.float32),
        pltpu.VMEM((C,), jnp.float32),
        pltpu.VMEM((C,), jnp.float32),
        pltpu.VMEM_SHARED((NP_, C), jnp.float32),
        pltpu.SemaphoreType.DMA,
    ],
    compiler_params=pltpu.CompilerParams(needs_layout_passes=False),
)
def _scatter(h_hbm, col3_hbm, sc_hbm, sh_hbm, zrow_hbm, zcnt_hbm,
             ssum_hbm, cnt_hbm,
             col2d, ha, hb, cntbuf, scbuf, shbuf, accum, sem_l):
    bufs = [ha, hb]
    cid = lax.axis_index("c")
    sid = lax.axis_index("s")
    wid = sid * NC + cid
    base = wid * EW

    pltpu.sync_copy(col3_hbm.at[wid], col2d)
    pltpu.sync_copy(sc_hbm, scbuf)
    pltpu.sync_copy(sh_hbm, shbuf)
    pltpu.sync_copy(zcnt_hbm, cntbuf)
    pltpu.sync_copy(zrow_hbm, accum.at[pl.ds(sid * STRIPE, STRIPE)])
    plsc.subcore_barrier()

    svs = [scbuf[pl.ds(g * 16, 16)] for g in range(8)]
    shs = [shbuf[pl.ds(g * 16, 16)] for g in range(8)]
    ones16 = jnp.full((16,), 1.0, jnp.float32)

    def fire(j, b):
        pltpu.async_copy(h_hbm.at[pl.ds(base + j * K, K)], bufs[b], sem_l)

    def slot(j, b, last):
        if not last:
            fire(j + 1, 1 - b)
        pltpu.make_async_copy(h_hbm.at[pl.ds(base, K)], bufs[b], sem_l).wait()

        def row_step(r4, c2):
            for dr in range(4):
                r = r4 * 4 + dr
                for g in range(8):
                    hv = bufs[b][r, pl.ds(g * 16, 16)]
                    yv = hv * svs[g] + shs[g]
                    yv = jnp.where(yv > 0.0, yv, jnp.exp(yv) - 1.0)
                    bufs[b][r, pl.ds(g * 16, 16)] = yv
            return c2

        lax.fori_loop(0, K // 4, row_step, 0)

        def cnt_step(t, c2):
            cv = col2d[j, pl.ds(t * 16, 16)]
            plsc.addupdate_scatter(cntbuf, [cv], ones16)
            return c2

        lax.fori_loop(0, K // 16, cnt_step, 0)

        pltpu.sync_copy(bufs[b], accum.at[col2d.at[j]], add=True)

    fire(jnp.int32(0), 0)

    def outer(j0):
        slot(j0, 0, False)
        slot(j0 + 1, 1, False)

    pl.loop(0, NCHUNK - 1, step=2)(outer)
    slot(jnp.int32(NCHUNK - 1), 0, True)

    plsc.subcore_barrier()
    pltpu.sync_copy(
        accum.at[pl.ds(sid * STRIPE, STRIPE)],
        ssum_hbm.at[cid, pl.ds(sid * STRIPE, STRIPE)],
    )
    pltpu.sync_copy(cntbuf, cnt_hbm.at[wid])


def _node_body(t2_ref, ss_ref, cnt_ref, w2_ref, b2_ref, g2_ref, be2_ref, o_ref):
    ssum = ss_ref[0, :N] + ss_ref[1, :N]
    cnt = jnp.sum(cnt_ref[...], axis=0)
    mean = ssum / jnp.clip(cnt, 1.0, None)[:, None]
    dn = (((1,), (1,)), ((), ()))
    o = (
        lax.dot_general(mean, w2_ref[:, C:], dn, preferred_element_type=jnp.float32)
        + t2_ref[...]
        + b2_ref[...]
    )
    m = jnp.mean(o, axis=0, keepdims=True)
    v = jnp.mean(o * o, axis=0, keepdims=True) - m * m
    y = (o - m) * lax.rsqrt(v + EPS) * g2_ref[...] + be2_ref[...]
    o_ref[...] = jnp.where(y > 0.0, y, jnp.exp(y) - 1.0)


def _node(t2, ssum, cnt, W2, b2, g2, be2):
    return pl.pallas_call(
        _node_body,
        out_shape=jax.ShapeDtypeStruct((N, C), jnp.float32),
    )(t2, ssum, cnt, W2, b2, g2, be2)


def kernel(x, edge_index, edge_attr, u, batch, W1, b1, g1, be1, W2, b2, g2, be2):
    row = edge_index[0]
    col3 = edge_index[1].reshape(NW, NCHUNK, K)
    t1, t2 = _pre(x, W1, b1.reshape(1, C), W2)
    G = _gather(t1, row)
    h, st = _edge(G, edge_attr, W1)
    m = st[0] / E
    var = st[1] / E - m * m
    scale = g1 / jnp.sqrt(var + EPS)
    shift = be1 - m * scale
    zrow = jnp.zeros((STRIPE, C), jnp.float32)
    zcnt = jnp.zeros((N,), jnp.float32)
    ssum, cnt = _scatter(h, col3, scale, shift, zrow, zcnt)
    return _node(
        t2, ssum, cnt, W2, b2.reshape(1, C), g2.reshape(1, C), be2.reshape(1, C)
    )


# speedup vs baseline: 2.8637x; 1.0141x over previous
import functools
import jax
import jax.numpy as jnp
from jax import lax
from jax.experimental import pallas as pl
from jax.experimental.pallas import tpu as pltpu
from jax.experimental.pallas import tpu_sc as plsc

N = 10000
E = 320000
C = 128
EPS = 1e-4

NC = 2
NS = 16
NW = NC * NS
E2 = E // 2
EWG = E2 // NW
KG = 40
NCG = EWG // KG
EW = E2 // NS
K = 80
NCHUNK = EW // K
NP_ = 10240
STRIPE = NP_ // NS

BE = 640
NBLK = E2 // BE


def _pre_body(x_ref, w1_ref, b1_ref, w2_ref, t1_ref, t2_ref):
    x = x_ref[...]
    dn = (((1,), (1,)), ((), ()))
    t1_ref[...] = (
        lax.dot_general(x, w1_ref[:, :C], dn, preferred_element_type=jnp.float32)
        + b1_ref[...]
    )
    t2_ref[...] = lax.dot_general(
        x, w2_ref[:, :C], dn, preferred_element_type=jnp.float32
    )


def _pre(x, W1, b1, W2):
    return pl.pallas_call(
        _pre_body,
        out_shape=[
            jax.ShapeDtypeStruct((N, C), jnp.float32),
            jax.ShapeDtypeStruct((N, C), jnp.float32),
        ],
    )(x, W1, b1, W2)


_MESH = plsc.VectorSubcoreMesh(core_axis_name="c", subcore_axis_name="s")


@functools.partial(
    pl.kernel,
    out_type=jax.ShapeDtypeStruct((E2, C), jnp.float32),
    mesh=_MESH,
    scratch_types=[
        pltpu.VMEM((KG, C), jnp.float32),
        pltpu.VMEM((KG, C), jnp.float32),
        pltpu.VMEM((EWG,), jnp.int32),
        pltpu.SemaphoreType.DMA,
        pltpu.SemaphoreType.DMA,
    ],
    compiler_params=pltpu.CompilerParams(needs_layout_passes=False),
)
def _gather(t1_hbm, row_hbm, g_hbm, rows_a, rows_b, idx_v, sem_g, sem_w):
    bufs = [rows_a, rows_b]
    cid = lax.axis_index("c")
    sid = lax.axis_index("s")
    wid = sid * NC + cid
    base = wid * EWG
    pltpu.sync_copy(row_hbm.at[pl.ds(base, EWG)], idx_v)

    def fire(j, b):
        pltpu.async_copy(t1_hbm.at[idx_v.at[pl.ds(j * KG, KG)]], bufs[b], sem_g)

    def slot(j, b, last):
        @pl.when(j >= 1)
        def _():
            pltpu.make_async_copy(bufs[1 - b], g_hbm.at[pl.ds(base, KG)], sem_w).wait()

        if not last:
            fire(j + 1, 1 - b)
        pltpu.make_async_copy(
            t1_hbm.at[idx_v.at[pl.ds(0, KG)]], bufs[b], sem_g
        ).wait()
        pltpu.async_copy(bufs[b], g_hbm.at[pl.ds(base + j * KG, KG)], sem_w)

    fire(jnp.int32(0), 0)

    def outer(j0):
        slot(j0, 0, False)
        slot(j0 + 1, 1, False)

    pl.loop(0, NCG - 1, step=2)(outer)
    slot(jnp.int32(NCG - 1), 0, True)
    pltpu.make_async_copy(bufs[0], g_hbm.at[pl.ds(base, KG)], sem_w).wait()


def _edge_body(g_ref, ea_ref, w1_ref, h_ref, st_ref):
    i = pl.program_id(0)
    dn = (((1,), (1,)), ((), ()))
    a = lax.dot_general(
        ea_ref[...], w1_ref[:, C:], dn, preferred_element_type=jnp.float32
    )
    h = a + g_ref[...]
    h_ref[...] = h
    s = jnp.sum(h, axis=0)
    q = jnp.sum(h * h, axis=0)
    upd = jnp.concatenate(
        [s[None], q[None], jnp.zeros((6, C), jnp.float32)], axis=0
    )

    @pl.when(i == 0)
    def _():
        st_ref[...] = upd

    @pl.when(i > 0)
    def _():
        st_ref[...] = st_ref[...] + upd


def _edge(G, edge_attr, W1):
    return pl.pallas_call(
        _edge_body,
        grid=(NBLK,),
        in_specs=[
            pl.BlockSpec((BE, C), lambda i: (i, 0)),
            pl.BlockSpec((BE, C), lambda i: (i, 0)),
            pl.BlockSpec((C, 2 * C), lambda i: (0, 0)),
        ],
        out_specs=[
            pl.BlockSpec((BE, C), lambda i: (i, 0)),
            pl.BlockSpec((8, C), lambda i: (0, 0)),
        ],
        out_shape=[
            jax.ShapeDtypeStruct((E2, C), jnp.float32),
            jax.ShapeDtypeStruct((8, C), jnp.float32),
        ],
    )(G, edge_attr, W1)


@functools.partial(
    pl.kernel,
    out_type=[
        jax.ShapeDtypeStruct((NC, NP_, C), jnp.float32),
        jax.ShapeDtypeStruct((NW, N), jnp.float32),
    ],
    mesh=_MESH,
    scratch_types=[
        pltpu.VMEM((NCHUNK, K), jnp.int32),
        pltpu.VMEM((K, C), jnp.float32),
        pltpu.VMEM((K, C), jnp.float32),
        pltpu.VMEM((N,), jnp.float32),
        pltpu.VMEM((C,), jnp.float32),
        pltpu.VMEM((C,), jnp.float32),
        pltpu.VMEM_SHARED((NP_, C), jnp.float32),
        pltpu.SemaphoreType.DMA,
    ],
    compiler_params=pltpu.CompilerParams(needs_layout_passes=False),
)
def _scatter(h0_hbm, h1_hbm, col4_hbm, sc_hbm, sh_hbm, zrow_hbm, zcnt_hbm,
             ssum_hbm, cnt_hbm,
             col2d, ha, hb, cntbuf, scbuf, shbuf, accum, sem_l):
    bufs = [ha, hb]
    cid = lax.axis_index("c")
    sid = lax.axis_index("s")
    wid = cid * NS + sid
    base = sid * EW

    pltpu.sync_copy(col4_hbm.at[cid, sid], col2d)
    pltpu.sync_copy(sc_hbm, scbuf)
    pltpu.sync_copy(sh_hbm, shbuf)
    pltpu.sync_copy(zcnt_hbm, cntbuf)
    pltpu.sync_copy(zrow_hbm, accum.at[pl.ds(sid * STRIPE, STRIPE)])
    plsc.subcore_barrier()

    svs = [scbuf[pl.ds(g * 16, 16)] for g in range(8)]
    shs = [shbuf[pl.ds(g * 16, 16)] for g in range(8)]
    ones16 = jnp.full((16,), 1.0, jnp.float32)

    def run(h_hbm):
        def fire(j, b):
            pltpu.async_copy(h_hbm.at[pl.ds(base + j * K, K)], bufs[b], sem_l)

        def slot(j, b, last):
            if not last:
                fire(j + 1, 1 - b)
            pltpu.make_async_copy(h_hbm.at[pl.ds(base, K)], bufs[b], sem_l).wait()

            def row_step(r4, c2):
                for dr in range(4):
                    r = r4 * 4 + dr
                    for g in range(8):
                        hv = bufs[b][r, pl.ds(g * 16, 16)]
                        yv = hv * svs[g] + shs[g]
                        yv = jnp.where(yv > 0.0, yv, jnp.exp(yv) - 1.0)
                        bufs[b][r, pl.ds(g * 16, 16)] = yv
                return c2

            lax.fori_loop(0, K // 4, row_step, 0)

            def cnt_step(t, c2):
                cv = col2d[j, pl.ds(t * 16, 16)]
                plsc.addupdate_scatter(cntbuf, [cv], ones16)
                return c2

            lax.fori_loop(0, K // 16, cnt_step, 0)

            pltpu.sync_copy(bufs[b], accum.at[col2d.at[j]], add=True)

        fire(jnp.int32(0), 0)

        def outer(j0):
            slot(j0, 0, False)
            slot(j0 + 1, 1, False)

        pl.loop(0, NCHUNK - 1, step=2)(outer)
        slot(jnp.int32(NCHUNK - 1), 0, True)

    @pl.when(cid == 0)
    def _():
        run(h0_hbm)

    @pl.when(cid == 1)
    def _():
        run(h1_hbm)

    plsc.subcore_barrier()
    pltpu.sync_copy(
        accum.at[pl.ds(sid * STRIPE, STRIPE)],
        ssum_hbm.at[cid, pl.ds(sid * STRIPE, STRIPE)],
    )
    pltpu.sync_copy(cntbuf, cnt_hbm.at[wid])


def _node_body(t2_ref, ss_ref, cnt_ref, w2_ref, b2_ref, g2_ref, be2_ref, o_ref):
    ssum = ss_ref[0, :N] + ss_ref[1, :N]
    cnt = jnp.sum(cnt_ref[...], axis=0)
    mean = ssum / jnp.clip(cnt, 1.0, None)[:, None]
    dn = (((1,), (1,)), ((), ()))
    o = (
        lax.dot_general(mean, w2_ref[:, C:], dn, preferred_element_type=jnp.float32)
        + t2_ref[...]
        + b2_ref[...]
    )
    m = jnp.mean(o, axis=0, keepdims=True)
    v = jnp.mean(o * o, axis=0, keepdims=True) - m * m
    y = (o - m) * lax.rsqrt(v + EPS) * g2_ref[...] + be2_ref[...]
    o_ref[...] = jnp.where(y > 0.0, y, jnp.exp(y) - 1.0)


def _node(t2, ssum, cnt, W2, b2, g2, be2):
    return pl.pallas_call(
        _node_body,
        out_shape=jax.ShapeDtypeStruct((N, C), jnp.float32),
    )(t2, ssum, cnt, W2, b2, g2, be2)


def kernel(x, edge_index, edge_attr, u, batch, W1, b1, g1, be1, W2, b2, g2, be2):
    row = edge_index[0]
    col4 = edge_index[1].reshape(NC, NS, NCHUNK, K)
    t1, t2 = _pre(x, W1, b1.reshape(1, C), W2)
    G0 = _gather(t1, row[:E2])
    h0, st0 = _edge(G0, edge_attr[:E2], W1)
    G1 = _gather(t1, row[E2:])
    h1, st1 = _edge(G1, edge_attr[E2:], W1)
    st = st0 + st1
    m = st[0] / E
    var = st[1] / E - m * m
    scale = g1 / jnp.sqrt(var + EPS)
    shift = be1 - m * scale
    zrow = jnp.zeros((STRIPE, C), jnp.float32)
    zcnt = jnp.zeros((N,), jnp.float32)
    ssum, cnt = _scatter(h0, h1, col4, scale, shift, zrow, zcnt)
    return _node(
        t2, ssum, cnt, W2, b2.reshape(1, C), g2.reshape(1, C), be2.reshape(1, C)
    )


# speedup vs baseline: 3.3851x; 1.1820x over previous
import functools
import jax
import jax.numpy as jnp
from jax import lax
from jax.experimental import pallas as pl
from jax.experimental.pallas import tpu as pltpu
from jax.experimental.pallas import tpu_sc as plsc

N = 10000
E = 320000
C = 128
EPS = 1e-4

NC = 2
NS = 16
NW = NC * NS
E2 = E // 2
EWG = E2 // NW
KG = 40
NCG = EWG // KG
EW = E2 // NS
K = 80
NCHUNK = EW // K
NP_ = 10240
STRIPE = NP_ // NS

BE = 1280
NBLK = E2 // BE


def _pre_body(x_ref, w1_ref, b1_ref, w2_ref, t1_ref, t2_ref):
    x = x_ref[...]
    dn = (((1,), (1,)), ((), ()))
    t1_ref[...] = (
        lax.dot_general(x, w1_ref[:, :C], dn, preferred_element_type=jnp.float32)
        + b1_ref[...]
    )
    t2_ref[...] = lax.dot_general(
        x, w2_ref[:, :C], dn, preferred_element_type=jnp.float32
    )


def _pre(x, W1, b1, W2):
    return pl.pallas_call(
        _pre_body,
        out_shape=[
            jax.ShapeDtypeStruct((N, C), jnp.float32),
            jax.ShapeDtypeStruct((N, C), jnp.float32),
        ],
    )(x, W1, b1, W2)


_MESH = plsc.VectorSubcoreMesh(core_axis_name="c", subcore_axis_name="s")


@functools.partial(
    pl.kernel,
    out_type=jax.ShapeDtypeStruct((E2, C), jnp.float32),
    mesh=_MESH,
    scratch_types=[
        pltpu.VMEM((KG, C), jnp.float32),
        pltpu.VMEM((KG, C), jnp.float32),
        pltpu.VMEM((EWG,), jnp.int32),
        pltpu.SemaphoreType.DMA,
        pltpu.SemaphoreType.DMA,
    ],
    compiler_params=pltpu.CompilerParams(needs_layout_passes=False),
)
def _gather(t1_hbm, row_hbm, g_hbm, rows_a, rows_b, idx_v, sem_g, sem_w):
    bufs = [rows_a, rows_b]
    cid = lax.axis_index("c")
    sid = lax.axis_index("s")
    wid = sid * NC + cid
    base = wid * EWG
    pltpu.sync_copy(row_hbm.at[pl.ds(base, EWG)], idx_v)

    def fire(j, b):
        pltpu.async_copy(t1_hbm.at[idx_v.at[pl.ds(j * KG, KG)]], bufs[b], sem_g)

    def slot(j, b, last):
        @pl.when(j >= 1)
        def _():
            pltpu.make_async_copy(bufs[1 - b], g_hbm.at[pl.ds(base, KG)], sem_w).wait()

        if not last:
            fire(j + 1, 1 - b)
        pltpu.make_async_copy(
            t1_hbm.at[idx_v.at[pl.ds(0, KG)]], bufs[b], sem_g
        ).wait()
        pltpu.async_copy(bufs[b], g_hbm.at[pl.ds(base + j * KG, KG)], sem_w)

    fire(jnp.int32(0), 0)

    def outer(j0):
        slot(j0, 0, False)
        slot(j0 + 1, 1, False)

    pl.loop(0, NCG - 1, step=2)(outer)
    slot(jnp.int32(NCG - 1), 0, True)
    pltpu.make_async_copy(bufs[0], g_hbm.at[pl.ds(base, KG)], sem_w).wait()


def _edge_body(g_ref, ea_ref, w1_ref, h_ref, st_ref):
    i = pl.program_id(0)
    dn = (((1,), (1,)), ((), ()))
    a = lax.dot_general(
        ea_ref[...], w1_ref[:, C:], dn, preferred_element_type=jnp.float32
    )
    h = a + g_ref[...]
    h_ref[...] = h
    s = jnp.sum(h, axis=0)
    q = jnp.sum(h * h, axis=0)
    upd = jnp.concatenate(
        [s[None], q[None], jnp.zeros((6, C), jnp.float32)], axis=0
    )

    @pl.when(i == 0)
    def _():
        st_ref[...] = upd

    @pl.when(i > 0)
    def _():
        st_ref[...] = st_ref[...] + upd


def _edge(G, edge_attr, W1):
    return pl.pallas_call(
        _edge_body,
        grid=(NBLK,),
        in_specs=[
            pl.BlockSpec((BE, C), lambda i: (i, 0)),
            pl.BlockSpec((BE, C), lambda i: (i, 0)),
            pl.BlockSpec((C, 2 * C), lambda i: (0, 0)),
        ],
        out_specs=[
            pl.BlockSpec((BE, C), lambda i: (i, 0)),
            pl.BlockSpec((8, C), lambda i: (0, 0)),
        ],
        out_shape=[
            jax.ShapeDtypeStruct((E2, C), jnp.float32),
            jax.ShapeDtypeStruct((8, C), jnp.float32),
        ],
    )(G, edge_attr, W1)


@functools.partial(
    pl.kernel,
    out_type=[
        jax.ShapeDtypeStruct((NC, NP_, C), jnp.float32),
        jax.ShapeDtypeStruct((NW, N), jnp.float32),
    ],
    mesh=_MESH,
    scratch_types=[
        pltpu.VMEM((NCHUNK, K), jnp.int32),
        pltpu.VMEM((K, C), jnp.float32),
        pltpu.VMEM((K, C), jnp.float32),
        pltpu.VMEM((N,), jnp.float32),
        pltpu.VMEM((C,), jnp.float32),
        pltpu.VMEM((C,), jnp.float32),
        pltpu.VMEM_SHARED((NP_, C), jnp.float32),
        pltpu.SemaphoreType.DMA,
    ],
    compiler_params=pltpu.CompilerParams(needs_layout_passes=False),
)
def _scatter(h0_hbm, h1_hbm, col4_hbm, sc_hbm, sh_hbm, zrow_hbm, zcnt_hbm,
             ssum_hbm, cnt_hbm,
             col2d, ha, hb, cntbuf, scbuf, shbuf, accum, sem_l):
    bufs = [ha, hb]
    cid = lax.axis_index("c")
    sid = lax.axis_index("s")
    wid = cid * NS + sid
    base = sid * EW

    pltpu.sync_copy(col4_hbm.at[cid, sid], col2d)
    pltpu.sync_copy(sc_hbm, scbuf)
    pltpu.sync_copy(sh_hbm, shbuf)
    pltpu.sync_copy(zcnt_hbm, cntbuf)
    pltpu.sync_copy(zrow_hbm, accum.at[pl.ds(sid * STRIPE, STRIPE)])
    plsc.subcore_barrier()

    svs = [scbuf[pl.ds(g * 16, 16)] for g in range(8)]
    shs = [shbuf[pl.ds(g * 16, 16)] for g in range(8)]
    ones16 = jnp.full((16,), 1.0, jnp.float32)

    def run(h_hbm):
        def fire(j, b):
            pltpu.async_copy(h_hbm.at[pl.ds(base + j * K, K)], bufs[b], sem_l)

        def slot(j, b, last):
            if not last:
                fire(j + 1, 1 - b)
            pltpu.make_async_copy(h_hbm.at[pl.ds(base, K)], bufs[b], sem_l).wait()

            def row_step(r4, c2):
                for dr in range(4):
                    r = r4 * 4 + dr
                    for g in range(8):
                        hv = bufs[b][r, pl.ds(g * 16, 16)]
                        yv = hv * svs[g] + shs[g]
                        yv = jnp.where(yv > 0.0, yv, jnp.exp(yv) - 1.0)
                        bufs[b][r, pl.ds(g * 16, 16)] = yv
                return c2

            lax.fori_loop(0, K // 4, row_step, 0)

            def cnt_step(t, c2):
                cv = col2d[j, pl.ds(t * 16, 16)]
                plsc.addupdate_scatter(cntbuf, [cv], ones16)
                return c2

            lax.fori_loop(0, K // 16, cnt_step, 0)

            pltpu.sync_copy(bufs[b], accum.at[col2d.at[j]], add=True)

        fire(jnp.int32(0), 0)

        def outer(j0):
            slot(j0, 0, False)
            slot(j0 + 1, 1, False)

        pl.loop(0, NCHUNK - 1, step=2)(outer)
        slot(jnp.int32(NCHUNK - 1), 0, True)

    @pl.when(cid == 0)
    def _():
        run(h0_hbm)

    @pl.when(cid == 1)
    def _():
        run(h1_hbm)

    plsc.subcore_barrier()
    pltpu.sync_copy(
        accum.at[pl.ds(sid * STRIPE, STRIPE)],
        ssum_hbm.at[cid, pl.ds(sid * STRIPE, STRIPE)],
    )
    pltpu.sync_copy(cntbuf, cnt_hbm.at[wid])


def _node_body(t2_ref, ss_ref, cnt_ref, w2_ref, b2_ref, g2_ref, be2_ref, o_ref):
    ssum = ss_ref[0, :N] + ss_ref[1, :N]
    cnt = jnp.sum(cnt_ref[...], axis=0)
    mean = ssum / jnp.clip(cnt, 1.0, None)[:, None]
    dn = (((1,), (1,)), ((), ()))
    o = (
        lax.dot_general(mean, w2_ref[:, C:], dn, preferred_element_type=jnp.float32)
        + t2_ref[...]
        + b2_ref[...]
    )
    m = jnp.mean(o, axis=0, keepdims=True)
    v = jnp.mean(o * o, axis=0, keepdims=True) - m * m
    y = (o - m) * lax.rsqrt(v + EPS) * g2_ref[...] + be2_ref[...]
    o_ref[...] = jnp.where(y > 0.0, y, jnp.exp(y) - 1.0)


def _node(t2, ssum, cnt, W2, b2, g2, be2):
    return pl.pallas_call(
        _node_body,
        out_shape=jax.ShapeDtypeStruct((N, C), jnp.float32),
    )(t2, ssum, cnt, W2, b2, g2, be2)


def kernel(x, edge_index, edge_attr, u, batch, W1, b1, g1, be1, W2, b2, g2, be2):
    row = edge_index[0]
    col4 = edge_index[1].reshape(NC, NS, NCHUNK, K)
    t1, t2 = _pre(x, W1, b1.reshape(1, C), W2)
    G0 = _gather(t1, row[:E2])
    h0, st0 = _edge(G0, edge_attr[:E2], W1)
    G1 = _gather(t1, row[E2:])
    h1, st1 = _edge(G1, edge_attr[E2:], W1)
    st = st0 + st1
    m = st[0] / E
    var = st[1] / E - m * m
    scale = g1 / jnp.sqrt(var + EPS)
    shift = be1 - m * scale
    zrow = jnp.zeros((STRIPE, C), jnp.float32)
    zcnt = jnp.zeros((N,), jnp.float32)
    ssum, cnt = _scatter(h0, h1, col4, scale, shift, zrow, zcnt)
    return _node(
        t2, ssum, cnt, W2, b2.reshape(1, C), g2.reshape(1, C), be2.reshape(1, C)
    )


# speedup vs baseline: 3.6456x; 1.0770x over previous
import functools
import jax
import jax.numpy as jnp
from jax import lax
from jax.experimental import pallas as pl
from jax.experimental.pallas import tpu as pltpu
from jax.experimental.pallas import tpu_sc as plsc

N = 10000
E = 320000
C = 128
EPS = 1e-4

NC = 2
NS = 16
NW = NC * NS
E2 = E // 2
EWG = E2 // NW
KG = 40
NCG = EWG // KG
EW = E2 // NS
K = 80
NCHUNK = EW // K
NP_ = 10240
STRIPE = NP_ // NS

BE = 2000
NBLK = E2 // BE


def _pre_body(x_ref, w1_ref, b1_ref, w2_ref, t1_ref, t2_ref):
    x = x_ref[...]
    dn = (((1,), (1,)), ((), ()))
    t1_ref[...] = (
        lax.dot_general(x, w1_ref[:, :C], dn, preferred_element_type=jnp.float32)
        + b1_ref[...]
    )
    t2_ref[...] = lax.dot_general(
        x, w2_ref[:, :C], dn, preferred_element_type=jnp.float32
    )


def _pre(x, W1, b1, W2):
    return pl.pallas_call(
        _pre_body,
        out_shape=[
            jax.ShapeDtypeStruct((N, C), jnp.float32),
            jax.ShapeDtypeStruct((N, C), jnp.float32),
        ],
    )(x, W1, b1, W2)


_MESH = plsc.VectorSubcoreMesh(core_axis_name="c", subcore_axis_name="s")


@functools.partial(
    pl.kernel,
    out_type=jax.ShapeDtypeStruct((E2, C), jnp.float32),
    mesh=_MESH,
    scratch_types=[
        pltpu.VMEM((KG, C), jnp.float32),
        pltpu.VMEM((KG, C), jnp.float32),
        pltpu.VMEM((EWG,), jnp.int32),
        pltpu.SemaphoreType.DMA,
        pltpu.SemaphoreType.DMA,
    ],
    compiler_params=pltpu.CompilerParams(needs_layout_passes=False),
)
def _gather(t1_hbm, row_hbm, g_hbm, rows_a, rows_b, idx_v, sem_g, sem_w):
    bufs = [rows_a, rows_b]
    cid = lax.axis_index("c")
    sid = lax.axis_index("s")
    wid = sid * NC + cid
    base = wid * EWG
    pltpu.sync_copy(row_hbm.at[pl.ds(base, EWG)], idx_v)

    def fire(j, b):
        pltpu.async_copy(t1_hbm.at[idx_v.at[pl.ds(j * KG, KG)]], bufs[b], sem_g)

    def slot(j, b, last):
        @pl.when(j >= 1)
        def _():
            pltpu.make_async_copy(bufs[1 - b], g_hbm.at[pl.ds(base, KG)], sem_w).wait()

        if not last:
            fire(j + 1, 1 - b)
        pltpu.make_async_copy(
            t1_hbm.at[idx_v.at[pl.ds(0, KG)]], bufs[b], sem_g
        ).wait()
        pltpu.async_copy(bufs[b], g_hbm.at[pl.ds(base + j * KG, KG)], sem_w)

    fire(jnp.int32(0), 0)

    def outer(j0):
        slot(j0, 0, False)
        slot(j0 + 1, 1, False)

    pl.loop(0, NCG - 1, step=2)(outer)
    slot(jnp.int32(NCG - 1), 0, True)
    pltpu.make_async_copy(bufs[0], g_hbm.at[pl.ds(base, KG)], sem_w).wait()


def _edge_body(g_ref, ea_ref, w1_ref, h_ref, st_ref):
    i = pl.program_id(0)
    dn = (((1,), (1,)), ((), ()))
    a = lax.dot_general(
        ea_ref[...], w1_ref[:, C:], dn, preferred_element_type=jnp.float32
    )
    h = a + g_ref[...]
    h_ref[...] = h
    s = jnp.sum(h, axis=0)
    q = jnp.sum(h * h, axis=0)
    upd = jnp.concatenate(
        [s[None], q[None], jnp.zeros((6, C), jnp.float32)], axis=0
    )

    @pl.when(i == 0)
    def _():
        st_ref[...] = upd

    @pl.when(i > 0)
    def _():
        st_ref[...] = st_ref[...] + upd


def _edge(G, edge_attr, W1):
    return pl.pallas_call(
        _edge_body,
        grid=(NBLK,),
        in_specs=[
            pl.BlockSpec((BE, C), lambda i: (i, 0)),
            pl.BlockSpec((BE, C), lambda i: (i, 0)),
            pl.BlockSpec((C, 2 * C), lambda i: (0, 0)),
        ],
        out_specs=[
            pl.BlockSpec((BE, C), lambda i: (i, 0)),
            pl.BlockSpec((8, C), lambda i: (0, 0)),
        ],
        out_shape=[
            jax.ShapeDtypeStruct((E2, C), jnp.float32),
            jax.ShapeDtypeStruct((8, C), jnp.float32),
        ],
    )(G, edge_attr, W1)


@functools.partial(
    pl.kernel,
    out_type=[
        jax.ShapeDtypeStruct((NC, NP_, C), jnp.float32),
        jax.ShapeDtypeStruct((NW, N), jnp.float32),
    ],
    mesh=_MESH,
    scratch_types=[
        pltpu.VMEM((NCHUNK, K), jnp.int32),
        pltpu.VMEM((K, C), jnp.float32),
        pltpu.VMEM((K, C), jnp.float32),
        pltpu.VMEM((N,), jnp.float32),
        pltpu.VMEM((C,), jnp.float32),
        pltpu.VMEM((C,), jnp.float32),
        pltpu.VMEM_SHARED((NP_, C), jnp.float32),
        pltpu.SemaphoreType.DMA,
    ],
    compiler_params=pltpu.CompilerParams(needs_layout_passes=False),
)
def _scatter(h0_hbm, h1_hbm, col4_hbm, sc_hbm, sh_hbm, zrow_hbm, zcnt_hbm,
             ssum_hbm, cnt_hbm,
             col2d, ha, hb, cntbuf, scbuf, shbuf, accum, sem_l):
    bufs = [ha, hb]
    cid = lax.axis_index("c")
    sid = lax.axis_index("s")
    wid = cid * NS + sid
    base = sid * EW

    pltpu.sync_copy(col4_hbm.at[cid, sid], col2d)
    pltpu.sync_copy(sc_hbm, scbuf)
    pltpu.sync_copy(sh_hbm, shbuf)
    pltpu.sync_copy(zcnt_hbm, cntbuf)
    pltpu.sync_copy(zrow_hbm, accum.at[pl.ds(sid * STRIPE, STRIPE)])
    plsc.subcore_barrier()

    svs = [scbuf[pl.ds(g * 16, 16)] for g in range(8)]
    shs = [shbuf[pl.ds(g * 16, 16)] for g in range(8)]
    ones16 = jnp.full((16,), 1.0, jnp.float32)

    def run(h_hbm):
        def fire(j, b):
            pltpu.async_copy(h_hbm.at[pl.ds(base + j * K, K)], bufs[b], sem_l)

        def slot(j, b, last):
            if not last:
                fire(j + 1, 1 - b)
            pltpu.make_async_copy(h_hbm.at[pl.ds(base, K)], bufs[b], sem_l).wait()

            def row_step(r4, c2):
                for dr in range(4):
                    r = r4 * 4 + dr
                    for g in range(8):
                        hv = bufs[b][r, pl.ds(g * 16, 16)]
                        yv = hv * svs[g] + shs[g]
                        yv = jnp.where(yv > 0.0, yv, jnp.exp(yv) - 1.0)
                        bufs[b][r, pl.ds(g * 16, 16)] = yv
                return c2

            lax.fori_loop(0, K // 4, row_step, 0)

            def cnt_step(t, c2):
                cv = col2d[j, pl.ds(t * 16, 16)]
                plsc.addupdate_scatter(cntbuf, [cv], ones16)
                return c2

            lax.fori_loop(0, K // 16, cnt_step, 0)

            pltpu.sync_copy(bufs[b], accum.at[col2d.at[j]], add=True)

        fire(jnp.int32(0), 0)

        def outer(j0):
            slot(j0, 0, False)
            slot(j0 + 1, 1, False)

        pl.loop(0, NCHUNK - 1, step=2)(outer)
        slot(jnp.int32(NCHUNK - 1), 0, True)

    @pl.when(cid == 0)
    def _():
        run(h0_hbm)

    @pl.when(cid == 1)
    def _():
        run(h1_hbm)

    plsc.subcore_barrier()
    pltpu.sync_copy(
        accum.at[pl.ds(sid * STRIPE, STRIPE)],
        ssum_hbm.at[cid, pl.ds(sid * STRIPE, STRIPE)],
    )
    pltpu.sync_copy(cntbuf, cnt_hbm.at[wid])


def _node_body(t2_ref, ss_ref, cnt_ref, w2_ref, b2_ref, g2_ref, be2_ref, o_ref):
    ssum = ss_ref[0, :N] + ss_ref[1, :N]
    cnt = jnp.sum(cnt_ref[...], axis=0)
    mean = ssum / jnp.clip(cnt, 1.0, None)[:, None]
    dn = (((1,), (1,)), ((), ()))
    o = (
        lax.dot_general(mean, w2_ref[:, C:], dn, preferred_element_type=jnp.float32)
        + t2_ref[...]
        + b2_ref[...]
    )
    m = jnp.mean(o, axis=0, keepdims=True)
    v = jnp.mean(o * o, axis=0, keepdims=True) - m * m
    y = (o - m) * lax.rsqrt(v + EPS) * g2_ref[...] + be2_ref[...]
    o_ref[...] = jnp.where(y > 0.0, y, jnp.exp(y) - 1.0)


def _node(t2, ssum, cnt, W2, b2, g2, be2):
    return pl.pallas_call(
        _node_body,
        out_shape=jax.ShapeDtypeStruct((N, C), jnp.float32),
    )(t2, ssum, cnt, W2, b2, g2, be2)


def kernel(x, edge_index, edge_attr, u, batch, W1, b1, g1, be1, W2, b2, g2, be2):
    row = edge_index[0]
    col4 = edge_index[1].reshape(NC, NS, NCHUNK, K)
    t1, t2 = _pre(x, W1, b1.reshape(1, C), W2)
    G0 = _gather(t1, row[:E2])
    h0, st0 = _edge(G0, edge_attr[:E2], W1)
    G1 = _gather(t1, row[E2:])
    h1, st1 = _edge(G1, edge_attr[E2:], W1)
    st = st0 + st1
    m = st[0] / E
    var = st[1] / E - m * m
    scale = g1 / jnp.sqrt(var + EPS)
    shift = be1 - m * scale
    zrow = jnp.zeros((STRIPE, C), jnp.float32)
    zcnt = jnp.zeros((N,), jnp.float32)
    ssum, cnt = _scatter(h0, h1, col4, scale, shift, zrow, zcnt)
    return _node(
        t2, ssum, cnt, W2, b2.reshape(1, C), g2.reshape(1, C), be2.reshape(1, C)
    )


# speedup vs baseline: 3.7765x; 1.0359x over previous
import functools
import jax
import jax.numpy as jnp
from jax import lax
from jax.experimental import pallas as pl
from jax.experimental.pallas import tpu as pltpu
from jax.experimental.pallas import tpu_sc as plsc

N = 10000
E = 320000
C = 128
EPS = 1e-4

NC = 2
NS = 16
NW = NC * NS
E2 = E // 2
EWG = E2 // NW
KG = 40
NCG = EWG // KG
EW = E2 // NS
K = 80
NCHUNK = EW // K
NP_ = 10240
STRIPE = NP_ // NS

BE = 4000
NBLK = E2 // BE


def _pre_body(x_ref, w1_ref, b1_ref, w2_ref, t1_ref, t2_ref):
    x = x_ref[...]
    dn = (((1,), (1,)), ((), ()))
    t1_ref[...] = (
        lax.dot_general(x, w1_ref[:, :C], dn, preferred_element_type=jnp.float32)
        + b1_ref[...]
    )
    t2_ref[...] = lax.dot_general(
        x, w2_ref[:, :C], dn, preferred_element_type=jnp.float32
    )


def _pre(x, W1, b1, W2):
    return pl.pallas_call(
        _pre_body,
        out_shape=[
            jax.ShapeDtypeStruct((N, C), jnp.float32),
            jax.ShapeDtypeStruct((N, C), jnp.float32),
        ],
    )(x, W1, b1, W2)


_MESH = plsc.VectorSubcoreMesh(core_axis_name="c", subcore_axis_name="s")


@functools.partial(
    pl.kernel,
    out_type=jax.ShapeDtypeStruct((E2, C), jnp.float32),
    mesh=_MESH,
    scratch_types=[
        pltpu.VMEM((KG, C), jnp.float32),
        pltpu.VMEM((KG, C), jnp.float32),
        pltpu.VMEM((EWG,), jnp.int32),
        pltpu.SemaphoreType.DMA,
        pltpu.SemaphoreType.DMA,
    ],
    compiler_params=pltpu.CompilerParams(needs_layout_passes=False),
)
def _gather(t1_hbm, row_hbm, g_hbm, rows_a, rows_b, idx_v, sem_g, sem_w):
    bufs = [rows_a, rows_b]
    cid = lax.axis_index("c")
    sid = lax.axis_index("s")
    wid = sid * NC + cid
    base = wid * EWG
    pltpu.sync_copy(row_hbm.at[pl.ds(base, EWG)], idx_v)

    def fire(j, b):
        pltpu.async_copy(t1_hbm.at[idx_v.at[pl.ds(j * KG, KG)]], bufs[b], sem_g)

    def slot(j, b, last):
        @pl.when(j >= 1)
        def _():
            pltpu.make_async_copy(bufs[1 - b], g_hbm.at[pl.ds(base, KG)], sem_w).wait()

        if not last:
            fire(j + 1, 1 - b)
        pltpu.make_async_copy(
            t1_hbm.at[idx_v.at[pl.ds(0, KG)]], bufs[b], sem_g
        ).wait()
        pltpu.async_copy(bufs[b], g_hbm.at[pl.ds(base + j * KG, KG)], sem_w)

    fire(jnp.int32(0), 0)

    def outer(j0):
        slot(j0, 0, False)
        slot(j0 + 1, 1, False)

    pl.loop(0, NCG - 1, step=2)(outer)
    slot(jnp.int32(NCG - 1), 0, True)
    pltpu.make_async_copy(bufs[0], g_hbm.at[pl.ds(base, KG)], sem_w).wait()


def _edge_body(g_ref, ea_ref, w1_ref, h_ref, st_ref):
    i = pl.program_id(0)
    dn = (((1,), (1,)), ((), ()))
    a = lax.dot_general(
        ea_ref[...], w1_ref[:, C:], dn, preferred_element_type=jnp.float32
    )
    h = a + g_ref[...]
    h_ref[...] = h
    s = jnp.sum(h, axis=0)
    q = jnp.sum(h * h, axis=0)
    upd = jnp.concatenate(
        [s[None], q[None], jnp.zeros((6, C), jnp.float32)], axis=0
    )

    @pl.when(i == 0)
    def _():
        st_ref[...] = upd

    @pl.when(i > 0)
    def _():
        st_ref[...] = st_ref[...] + upd


def _edge(G, edge_attr, W1):
    return pl.pallas_call(
        _edge_body,
        grid=(NBLK,),
        in_specs=[
            pl.BlockSpec((BE, C), lambda i: (i, 0)),
            pl.BlockSpec((BE, C), lambda i: (i, 0)),
            pl.BlockSpec((C, 2 * C), lambda i: (0, 0)),
        ],
        out_specs=[
            pl.BlockSpec((BE, C), lambda i: (i, 0)),
            pl.BlockSpec((8, C), lambda i: (0, 0)),
        ],
        out_shape=[
            jax.ShapeDtypeStruct((E2, C), jnp.float32),
            jax.ShapeDtypeStruct((8, C), jnp.float32),
        ],
    )(G, edge_attr, W1)


@functools.partial(
    pl.kernel,
    out_type=[
        jax.ShapeDtypeStruct((NC, NP_, C), jnp.float32),
        jax.ShapeDtypeStruct((NW, N), jnp.float32),
    ],
    mesh=_MESH,
    scratch_types=[
        pltpu.VMEM((NCHUNK, K), jnp.int32),
        pltpu.VMEM((K, C), jnp.float32),
        pltpu.VMEM((K, C), jnp.float32),
        pltpu.VMEM((N,), jnp.float32),
        pltpu.VMEM((C,), jnp.float32),
        pltpu.VMEM((C,), jnp.float32),
        pltpu.VMEM_SHARED((NP_, C), jnp.float32),
        pltpu.SemaphoreType.DMA,
    ],
    compiler_params=pltpu.CompilerParams(needs_layout_passes=False),
)
def _scatter(h0_hbm, h1_hbm, col4_hbm, sc_hbm, sh_hbm, zrow_hbm, zcnt_hbm,
             ssum_hbm, cnt_hbm,
             col2d, ha, hb, cntbuf, scbuf, shbuf, accum, sem_l):
    bufs = [ha, hb]
    cid = lax.axis_index("c")
    sid = lax.axis_index("s")
    wid = cid * NS + sid
    base = sid * EW

    pltpu.sync_copy(col4_hbm.at[cid, sid], col2d)
    pltpu.sync_copy(sc_hbm, scbuf)
    pltpu.sync_copy(sh_hbm, shbuf)
    pltpu.sync_copy(zcnt_hbm, cntbuf)
    pltpu.sync_copy(zrow_hbm, accum.at[pl.ds(sid * STRIPE, STRIPE)])
    plsc.subcore_barrier()

    svs = [scbuf[pl.ds(g * 16, 16)] for g in range(8)]
    shs = [shbuf[pl.ds(g * 16, 16)] for g in range(8)]
    ones16 = jnp.full((16,), 1.0, jnp.float32)

    def run(h_hbm):
        def fire(j, b):
            pltpu.async_copy(h_hbm.at[pl.ds(base + j * K, K)], bufs[b], sem_l)

        def slot(j, b, last):
            if not last:
                fire(j + 1, 1 - b)
            pltpu.make_async_copy(h_hbm.at[pl.ds(base, K)], bufs[b], sem_l).wait()

            def row_step(r4, c2):
                for dr in range(4):
                    r = r4 * 4 + dr
                    for g in range(8):
                        hv = bufs[b][r, pl.ds(g * 16, 16)]
                        yv = hv * svs[g] + shs[g]
                        yv = jnp.where(yv > 0.0, yv, jnp.exp(yv) - 1.0)
                        bufs[b][r, pl.ds(g * 16, 16)] = yv
                return c2

            lax.fori_loop(0, K // 4, row_step, 0)

            def cnt_step(t, c2):
                cv = col2d[j, pl.ds(t * 16, 16)]
                plsc.addupdate_scatter(cntbuf, [cv], ones16)
                return c2

            lax.fori_loop(0, K // 16, cnt_step, 0)

            pltpu.sync_copy(bufs[b], accum.at[col2d.at[j]], add=True)

        fire(jnp.int32(0), 0)

        def outer(j0):
            slot(j0, 0, False)
            slot(j0 + 1, 1, False)

        pl.loop(0, NCHUNK - 1, step=2)(outer)
        slot(jnp.int32(NCHUNK - 1), 0, True)

    @pl.when(cid == 0)
    def _():
        run(h0_hbm)

    @pl.when(cid == 1)
    def _():
        run(h1_hbm)

    plsc.subcore_barrier()
    pltpu.sync_copy(
        accum.at[pl.ds(sid * STRIPE, STRIPE)],
        ssum_hbm.at[cid, pl.ds(sid * STRIPE, STRIPE)],
    )
    pltpu.sync_copy(cntbuf, cnt_hbm.at[wid])


def _node_body(t2_ref, ss_ref, cnt_ref, w2_ref, b2_ref, g2_ref, be2_ref, o_ref):
    ssum = ss_ref[0, :N] + ss_ref[1, :N]
    cnt = jnp.sum(cnt_ref[...], axis=0)
    mean = ssum / jnp.clip(cnt, 1.0, None)[:, None]
    dn = (((1,), (1,)), ((), ()))
    o = (
        lax.dot_general(mean, w2_ref[:, C:], dn, preferred_element_type=jnp.float32)
        + t2_ref[...]
        + b2_ref[...]
    )
    m = jnp.mean(o, axis=0, keepdims=True)
    v = jnp.mean(o * o, axis=0, keepdims=True) - m * m
    y = (o - m) * lax.rsqrt(v + EPS) * g2_ref[...] + be2_ref[...]
    o_ref[...] = jnp.where(y > 0.0, y, jnp.exp(y) - 1.0)


def _node(t2, ssum, cnt, W2, b2, g2, be2):
    return pl.pallas_call(
        _node_body,
        out_shape=jax.ShapeDtypeStruct((N, C), jnp.float32),
    )(t2, ssum, cnt, W2, b2, g2, be2)


def kernel(x, edge_index, edge_attr, u, batch, W1, b1, g1, be1, W2, b2, g2, be2):
    row = edge_index[0]
    col4 = edge_index[1].reshape(NC, NS, NCHUNK, K)
    t1, t2 = _pre(x, W1, b1.reshape(1, C), W2)
    G0 = _gather(t1, row[:E2])
    h0, st0 = _edge(G0, edge_attr[:E2], W1)
    G1 = _gather(t1, row[E2:])
    h1, st1 = _edge(G1, edge_attr[E2:], W1)
    st = st0 + st1
    m = st[0] / E
    var = st[1] / E - m * m
    scale = g1 / jnp.sqrt(var + EPS)
    shift = be1 - m * scale
    zrow = jnp.zeros((STRIPE, C), jnp.float32)
    zcnt = jnp.zeros((N,), jnp.float32)
    ssum, cnt = _scatter(h0, h1, col4, scale, shift, zrow, zcnt)
    return _node(
        t2, ssum, cnt, W2, b2.reshape(1, C), g2.reshape(1, C), be2.reshape(1, C)
    )
